# baseline (speedup 1.0000x reference)
import functools

import jax
import jax.numpy as jnp
from jax import lax
from jax.experimental import pallas as pl
from jax.experimental.pallas import tpu as pltpu
from jax.experimental.pallas import tpu_sc as plsc

_N, _E, _DIN, _DOUT, _G, _A = 10000, 320000, 128, 64, 256, 256

_NC, _NS = 2, 16
_NW = _NC * _NS
_EPW = _E // _NW
_CH = 400
_NCHUNK = _EPW // _CH
_NPAD = 10240
_NPT = _NPAD // _NS

_BN = 1000
_NBLK = _N // _BN


def _mm_body(x_ref, w_ref, o_ref):
    o_ref[...] = jnp.dot(x_ref[...], w_ref[...],
                         preferred_element_type=jnp.float32)


def _node_matmul(x, w):
    return pl.pallas_call(
        _mm_body,
        grid=(_NBLK,),
        in_specs=[pl.BlockSpec((_BN, _DIN), lambda i: (i, 0)),
                  pl.BlockSpec((_DIN, _DOUT), lambda i: (0, 0))],
        out_specs=pl.BlockSpec((_BN, _DOUT), lambda i: (i, 0)),
        out_shape=jax.ShapeDtypeStruct((_N, _DOUT), jnp.float32),
    )(x, w)


def _sc_scatter(h, ei4d, w):
    mesh = plsc.VectorSubcoreMesh(core_axis_name="c", subcore_axis_name="s")

    @functools.partial(
        pl.kernel,
        mesh=mesh,
        out_type=jax.ShapeDtypeStruct((_NC, _NPAD, _DOUT), jnp.float32),
        compiler_params=pltpu.CompilerParams(use_tc_tiling_on_sc=False),
        scratch_types=[
            pltpu.VMEM((_NCHUNK, _CH), jnp.int32),
            pltpu.VMEM((_NCHUNK, _CH), jnp.int32),
            pltpu.VMEM((_EPW,), jnp.float32),
            pltpu.VMEM((2, _CH, _DOUT), jnp.float32),
            pltpu.VMEM((_NPT // 10, _DOUT), jnp.float32),
            pltpu.VMEM_SHARED((_NPAD, _DOUT), jnp.float32),
            pltpu.SemaphoreType.DMA,
            pltpu.SemaphoreType.DMA,
            pltpu.SemaphoreType.DMA,
            pltpu.SemaphoreType.DMA,
        ],
    )
    def sc_kernel(h_hbm, ei_hbm, w_hbm, out_hbm,
                  src_v, dst_v, w_v, rows, zero_v, agg_sh,
                  g0, g1, s0, s1):
        cid = lax.axis_index("c")
        sid = lax.axis_index("s")
        wid = sid * _NC + cid

        pltpu.async_copy(ei_hbm.at[wid], src_v, g0)
        pltpu.async_copy(ei_hbm.at[_NW + wid], dst_v, g1)
        pltpu.async_copy(w_hbm.at[wid], w_v, s0)

        z16 = jnp.zeros((16,), jnp.float32)
        zrows = _NPT // 10

        def _zero_row(r, c):
            for q in range(_DOUT // 16):
                zero_v[r, pl.ds(q * 16, 16)] = z16
            return c

        lax.fori_loop(0, zrows, _zero_row, 0)

        def _zero_block(z, c):
            pltpu.async_copy(
                zero_v, agg_sh.at[pl.ds(sid * _NPT + z * zrows, zrows)], s1)
            return c

        lax.fori_loop(0, 10, _zero_block, 0)

        def _zero_drain(z, c):
            pltpu.make_async_copy(
                zero_v, agg_sh.at[pl.ds(sid * _NPT + z * zrows, zrows)],
                s1).wait()
            return c

        lax.fori_loop(0, 10, _zero_drain, 0)
        pltpu.make_async_copy(ei_hbm.at[wid], src_v, g0).wait()
        pltpu.make_async_copy(ei_hbm.at[_NW + wid], dst_v, g1).wait()
        pltpu.make_async_copy(w_hbm.at[wid], w_v, s0).wait()

        plsc.subcore_barrier()

        def _scale(i, buf):
            def _group(jb, c):
                w16 = w_v[pl.ds(i * _CH + jb * 16, 16)]
                for t in range(16):
                    wj = lax.gather(
                        w16, jnp.full((16, 1), t, jnp.int32),
                        lax.GatherDimensionNumbers(
                            offset_dims=(), collapsed_slice_dims=(0,),
                            start_index_map=(0,)),
                        slice_sizes=(1,),
                        mode=lax.GatherScatterMode.PROMISE_IN_BOUNDS)
                    j = jb * 16 + t
                    for q in range(_DOUT // 16):
                        sl = pl.ds(q * 16, 16)
                        buf[j, sl] = buf[j, sl] * wj
                return c

            lax.fori_loop(0, _CH // 16, _group, 0)

        def _gather(i, b, sem):
            pltpu.async_copy(h_hbm.at[src_v.at[i]], rows.at[b], sem)

        def _gather_wait(i, b, sem):
            pltpu.make_async_copy(h_hbm.at[src_v.at[i]], rows.at[b],
                                  sem).wait()

        def _scatter(i, b, sem):
            pltpu.async_copy(rows.at[b], agg_sh.at[dst_v.at[i]], sem,
                             add=True)

        def _scatter_wait(i, b, sem):
            pltpu.make_async_copy(rows.at[b], agg_sh.at[dst_v.at[i]],
                                  sem).wait()

        last = _NCHUNK - 1
        _gather(0, 0, g0)
        _gather(1, 1, g1)

        def _pair(k, c):
            i = k * 2
            _gather_wait(i, 0, g0)
            _scale(i, rows.at[0])
            _scatter(i, 0, s0)
            _gather_wait(i + 1, 1, g1)
            _scale(i + 1, rows.at[1])
            _scatter(i + 1, 1, s1)
            _scatter_wait(i, 0, s0)
            _gather(jnp.minimum(i + 2, last), 0, g0)
            _scatter_wait(i + 1, 1, s1)
            _gather(jnp.minimum(i + 3, last), 1, g1)
            return c

        lax.fori_loop(0, _NCHUNK // 2, _pair, 0)
        _gather_wait(last, 0, g0)
        _scale(last, rows.at[0])
        pltpu.sync_copy(rows.at[0], agg_sh.at[dst_v.at[last]], add=True)
        _gather_wait(last, 1, g1)

        plsc.subcore_barrier()
        pltpu.sync_copy(
            agg_sh.at[pl.ds(sid * _NPT, _NPT)],
            out_hbm.at[cid, pl.ds(sid * _NPT, _NPT)])

    return sc_kernel(h, ei4d, w)


def _pool_body(gid_ref, p0_ref, p1_ref, anc_ref, w_ref, b_ref, a_ref,
               hout_ref, pool_ref, anc_out_ref, acc_ref, cnt_ref):
    i = pl.program_id(0)
    x = p0_ref[0] + p1_ref[0] + b_ref[...]
    h = jnp.maximum(x, 0.0) + a_ref[...] * jnp.minimum(x, 0.0)
    hout_ref[...] = h

    oh = (gid_ref[0] == lax.broadcasted_iota(jnp.int32, (_G, _BN), 0)
          ).astype(jnp.float32)

    @pl.when(i == 0)
    def _():
        acc_ref[...] = jnp.zeros_like(acc_ref)
        cnt_ref[...] = jnp.zeros_like(cnt_ref)
        ah = jnp.dot(anc_ref[...], w_ref[...],
                     preferred_element_type=jnp.float32) + b_ref[...]
        anc_out_ref[...] = (jnp.maximum(ah, 0.0)
                            + a_ref[...] * jnp.minimum(ah, 0.0))

    acc_ref[...] += jnp.dot(oh, h, preferred_element_type=jnp.float32)
    cnt_ref[...] += jnp.sum(oh, axis=1, keepdims=True)

    @pl.when(i == _NBLK - 1)
    def _():
        pool_ref[...] = acc_ref[...] / jnp.maximum(cnt_ref[...], 1.0)


def _combine_pool(gid3, p0, p1, anc, w, b2, a2):
    return pl.pallas_call(
        _pool_body,
        grid=(_NBLK,),
        in_specs=[
            pl.BlockSpec((1, 1, _BN), lambda i: (i, 0, 0)),
            pl.BlockSpec((1, _BN, _DOUT), lambda i: (0, i, 0)),
            pl.BlockSpec((1, _BN, _DOUT), lambda i: (1, i, 0)),
            pl.BlockSpec((_A, _DIN), lambda i: (0, 0)),
            pl.BlockSpec((_DIN, _DOUT), lambda i: (0, 0)),
            pl.BlockSpec((1, _DOUT), lambda i: (0, 0)),
            pl.BlockSpec((1, 1), lambda i: (0, 0)),
        ],
        out_specs=[
            pl.BlockSpec((_BN, _DOUT), lambda i: (i, 0)),
            pl.BlockSpec((_G, _DOUT), lambda i: (0, 0)),
            pl.BlockSpec((_A, _DOUT), lambda i: (0, 0)),
        ],
        out_shape=[jax.ShapeDtypeStruct((_N, _DOUT), jnp.float32),
                   jax.ShapeDtypeStruct((_G, _DOUT), jnp.float32),
                   jax.ShapeDtypeStruct((_A, _DOUT), jnp.float32)],
        scratch_shapes=[pltpu.VMEM((_G, _DOUT), jnp.float32),
                        pltpu.VMEM((_G, 1), jnp.float32)],
    )(gid3, p0, p1, anc, w, b2, a2)


def kernel(in_feat, edge_index, edge_weight, graph_ids, anchor_embs,
           W, bias, prelu_a):
    b2 = bias.reshape(1, _DOUT)
    a2 = prelu_a.reshape(1, 1)
    h = _node_matmul(in_feat, W)
    ei4d = edge_index.astype(jnp.int32).reshape(2 * _NW, _NCHUNK, _CH)
    w2d = edge_weight.reshape(_NW, _EPW)
    partial = _sc_scatter(h, ei4d, w2d)
    gid3 = graph_ids.astype(jnp.int32).reshape(_NBLK, 1, _BN)
    h_out, pool, anchor_out = _combine_pool(
        gid3, partial, partial, anchor_embs, W, b2, a2)
    return (h_out, pool, anchor_out)


# speedup vs baseline: 1.4779x; 1.4779x over previous
import functools

import jax
import jax.numpy as jnp
from jax import lax
from jax.experimental import pallas as pl
from jax.experimental.pallas import tpu as pltpu
from jax.experimental.pallas import tpu_sc as plsc

_N, _E, _DIN, _DOUT, _G, _A = 10000, 320000, 128, 64, 256, 256

_NC, _NS = 2, 16
_NW = _NC * _NS
_EPW = _E // _NW
_CH = 400
_NCHUNK = _EPW // _CH
_SUB = 80
_NPAD = 10240
_NPT = _NPAD // _NS

_BN = 1000
_NBLK = _N // _BN


def _mm_body(x_ref, w_ref, o_ref):
    o_ref[...] = jnp.dot(x_ref[...], w_ref[...],
                         preferred_element_type=jnp.float32)


def _node_matmul(x, w):
    return pl.pallas_call(
        _mm_body,
        grid=(_NBLK,),
        in_specs=[pl.BlockSpec((_BN, _DIN), lambda i: (i, 0)),
                  pl.BlockSpec((_DIN, _DOUT), lambda i: (0, 0))],
        out_specs=pl.BlockSpec((_BN, _DOUT), lambda i: (i, 0)),
        out_shape=jax.ShapeDtypeStruct((_N, _DOUT), jnp.float32),
    )(x, w)


def _sc_scatter(h, ei4d, w):
    mesh = plsc.VectorSubcoreMesh(core_axis_name="c", subcore_axis_name="s")

    @functools.partial(
        pl.kernel,
        mesh=mesh,
        out_type=jax.ShapeDtypeStruct((_NC, _NPAD, _DOUT), jnp.float32),
        compiler_params=pltpu.CompilerParams(use_tc_tiling_on_sc=False),
        scratch_types=[
            pltpu.VMEM((_NCHUNK, _CH), jnp.int32),
            pltpu.VMEM((_NCHUNK, _CH), jnp.int32),
            pltpu.VMEM((_EPW,), jnp.float32),
            pltpu.VMEM((2, _CH, _DOUT), jnp.float32),
            pltpu.VMEM((_NPT // 10, _DOUT), jnp.float32),
            pltpu.VMEM_SHARED((_NPAD, _DOUT), jnp.float32),
            pltpu.SemaphoreType.DMA,
            pltpu.SemaphoreType.DMA,
            pltpu.SemaphoreType.DMA,
            pltpu.SemaphoreType.DMA,
        ],
    )
    def sc_kernel(h_hbm, ei_hbm, w_hbm, out_hbm,
                  src_v, dst_v, w_v, rows, zero_v, agg_sh,
                  g0, g1, s0, s1):
        cid = lax.axis_index("c")
        sid = lax.axis_index("s")
        wid = sid * _NC + cid

        pltpu.async_copy(ei_hbm.at[wid], src_v, g0)
        pltpu.async_copy(ei_hbm.at[_NW + wid], dst_v, g1)
        pltpu.async_copy(w_hbm.at[wid], w_v, s0)

        z16 = jnp.zeros((16,), jnp.float32)
        zrows = _NPT // 10

        def _zero_row(r, c):
            for q in range(_DOUT // 16):
                zero_v[r, pl.ds(q * 16, 16)] = z16
            return c

        lax.fori_loop(0, zrows, _zero_row, 0)

        def _zero_block(z, c):
            pltpu.async_copy(
                zero_v, agg_sh.at[pl.ds(sid * _NPT + z * zrows, zrows)], s1)
            return c

        lax.fori_loop(0, 10, _zero_block, 0)

        def _zero_drain(z, c):
            pltpu.make_async_copy(
                zero_v, agg_sh.at[pl.ds(sid * _NPT + z * zrows, zrows)],
                s1).wait()
            return c

        lax.fori_loop(0, 10, _zero_drain, 0)
        pltpu.make_async_copy(ei_hbm.at[wid], src_v, g0).wait()
        pltpu.make_async_copy(ei_hbm.at[_NW + wid], dst_v, g1).wait()
        pltpu.make_async_copy(w_hbm.at[wid], w_v, s0).wait()

        plsc.subcore_barrier()

        def _scale(i, buf):
            def _sub(z, c):
                base = z * _SUB
                for jb in range(_SUB // 16):
                    w16 = w_v[pl.ds(i * _CH + z * _SUB + jb * 16, 16)]
                    for t in range(16):
                        wj = lax.gather(
                            w16, jnp.full((16, 1), t, jnp.int32),
                            lax.GatherDimensionNumbers(
                                offset_dims=(), collapsed_slice_dims=(0,),
                                start_index_map=(0,)),
                            slice_sizes=(1,),
                            mode=lax.GatherScatterMode.PROMISE_IN_BOUNDS)
                        j = jb * 16 + t
                        for q in range(_DOUT // 16):
                            sl = pl.ds(q * 16, 16)
                            buf[base + j, sl] = buf[base + j, sl] * wj
                return c

            lax.fori_loop(0, _CH // _SUB, _sub, 0)

        def _gather(i, b, sem):
            pltpu.async_copy(h_hbm.at[src_v.at[i]], rows.at[b], sem)

        def _gather_wait(i, b, sem):
            pltpu.make_async_copy(h_hbm.at[src_v.at[i]], rows.at[b],
                                  sem).wait()

        def _scatter(i, b, sem):
            pltpu.async_copy(rows.at[b], agg_sh.at[dst_v.at[i]], sem,
                             add=True)

        def _scatter_wait(i, b, sem):
            pltpu.make_async_copy(rows.at[b], agg_sh.at[dst_v.at[i]],
                                  sem).wait()

        last = _NCHUNK - 1
        _gather(0, 0, g0)
        _gather(1, 1, g1)

        def _pair(k, c):
            i = k * 2
            _gather_wait(i, 0, g0)
            _scale(i, rows.at[0])
            _scatter(i, 0, s0)
            _gather_wait(i + 1, 1, g1)
            _scale(i + 1, rows.at[1])
            _scatter(i + 1, 1, s1)
            _scatter_wait(i, 0, s0)
            _gather(jnp.minimum(i + 2, last), 0, g0)
            _scatter_wait(i + 1, 1, s1)
            _gather(jnp.minimum(i + 3, last), 1, g1)
            return c

        lax.fori_loop(0, _NCHUNK // 2, _pair, 0)
        _gather_wait(last, 0, g0)
        _scale(last, rows.at[0])
        pltpu.sync_copy(rows.at[0], agg_sh.at[dst_v.at[last]], add=True)
        _gather_wait(last, 1, g1)

        plsc.subcore_barrier()
        pltpu.sync_copy(
            agg_sh.at[pl.ds(sid * _NPT, _NPT)],
            out_hbm.at[cid, pl.ds(sid * _NPT, _NPT)])

    return sc_kernel(h, ei4d, w)


def _pool_body(gid_ref, p0_ref, p1_ref, anc_ref, w_ref, b_ref, a_ref,
               hout_ref, pool_ref, anc_out_ref, acc_ref, cnt_ref):
    i = pl.program_id(0)
    x = p0_ref[0] + p1_ref[0] + b_ref[...]
    h = jnp.maximum(x, 0.0) + a_ref[...] * jnp.minimum(x, 0.0)
    hout_ref[...] = h

    oh = (gid_ref[0] == lax.broadcasted_iota(jnp.int32, (_G, _BN), 0)
          ).astype(jnp.float32)

    @pl.when(i == 0)
    def _():
        acc_ref[...] = jnp.zeros_like(acc_ref)
        cnt_ref[...] = jnp.zeros_like(cnt_ref)
        ah = jnp.dot(anc_ref[...], w_ref[...],
                     preferred_element_type=jnp.float32) + b_ref[...]
        anc_out_ref[...] = (jnp.maximum(ah, 0.0)
                            + a_ref[...] * jnp.minimum(ah, 0.0))

    acc_ref[...] += jnp.dot(oh, h, preferred_element_type=jnp.float32)
    cnt_ref[...] += jnp.sum(oh, axis=1, keepdims=True)

    @pl.when(i == _NBLK - 1)
    def _():
        pool_ref[...] = acc_ref[...] / jnp.maximum(cnt_ref[...], 1.0)


def _combine_pool(gid3, p0, p1, anc, w, b2, a2):
    return pl.pallas_call(
        _pool_body,
        grid=(_NBLK,),
        in_specs=[
            pl.BlockSpec((1, 1, _BN), lambda i: (i, 0, 0)),
            pl.BlockSpec((1, _BN, _DOUT), lambda i: (0, i, 0)),
            pl.BlockSpec((1, _BN, _DOUT), lambda i: (1, i, 0)),
            pl.BlockSpec((_A, _DIN), lambda i: (0, 0)),
            pl.BlockSpec((_DIN, _DOUT), lambda i: (0, 0)),
            pl.BlockSpec((1, _DOUT), lambda i: (0, 0)),
            pl.BlockSpec((1, 1), lambda i: (0, 0)),
        ],
        out_specs=[
            pl.BlockSpec((_BN, _DOUT), lambda i: (i, 0)),
            pl.BlockSpec((_G, _DOUT), lambda i: (0, 0)),
            pl.BlockSpec((_A, _DOUT), lambda i: (0, 0)),
        ],
        out_shape=[jax.ShapeDtypeStruct((_N, _DOUT), jnp.float32),
                   jax.ShapeDtypeStruct((_G, _DOUT), jnp.float32),
                   jax.ShapeDtypeStruct((_A, _DOUT), jnp.float32)],
        scratch_shapes=[pltpu.VMEM((_G, _DOUT), jnp.float32),
                        pltpu.VMEM((_G, 1), jnp.float32)],
    )(gid3, p0, p1, anc, w, b2, a2)


def kernel(in_feat, edge_index, edge_weight, graph_ids, anchor_embs,
           W, bias, prelu_a):
    b2 = bias.reshape(1, _DOUT)
    a2 = prelu_a.reshape(1, 1)
    h = _node_matmul(in_feat, W)
    ei4d = edge_index.astype(jnp.int32).reshape(2 * _NW, _NCHUNK, _CH)
    w2d = edge_weight.reshape(_NW, _EPW)
    partial = _sc_scatter(h, ei4d, w2d)
    gid3 = graph_ids.astype(jnp.int32).reshape(_NBLK, 1, _BN)
    h_out, pool, anchor_out = _combine_pool(
        gid3, partial, partial, anchor_embs, W, b2, a2)
    return (h_out, pool, anchor_out)
